# speedup vs baseline: 4.9356x; 4.9356x over previous
import jax
import jax.numpy as jnp
from jax.experimental import pallas as pl


def _tc_body(ids_ref, emb_ref, sp_ref, out_ref):
    ids = ids_ref[...][..., None]
    e0 = emb_ref[0]
    e1 = emb_ref[1]
    e2 = emb_ref[2]
    sp = sp_ref[...][None, :, None, :]
    out_ref[...] = jnp.where(ids == 0, e0, jnp.where(ids == 1, e1, e2)) + sp


def kernel(input_ids, embedding, species_embedding):
    B, S, T = input_ids.shape
    H = embedding.shape[1]
    return pl.pallas_call(
        _tc_body,
        grid=(B,),
        in_specs=[
            pl.BlockSpec((1, S, T), lambda b: (b, 0, 0)),
            pl.BlockSpec((3, H), lambda b: (0, 0)),
            pl.BlockSpec((S, H), lambda b: (0, 0)),
        ],
        out_specs=pl.BlockSpec((1, S, T, H), lambda b: (b, 0, 0, 0)),
        out_shape=jax.ShapeDtypeStruct((B, S, T, H), jnp.float32),
    )(input_ids, embedding, species_embedding)
